# baseline (speedup 1.0000x reference)
import functools

import jax
import jax.numpy as jnp
from jax import lax
from jax.experimental import pallas as pl
from jax.experimental.pallas import tpu as pltpu
from jax.experimental.pallas import tpu_sc as plsc

N_STOCKS = 100000
N_SECTORS = 20
D_MODEL = 64
BATCH = 16384

_NC = 2
_NS = 16
_NW = _NC * _NS
_DPW = D_MODEL // _NW
_BBLK = 2048
_NBLK = BATCH // _BBLK


def _emb_body(stock_ids_hbm, sector_ids_hbm, stock_tabT_hbm, sector_tabT_hbm,
              outT_hbm, row_v, secrow_v, sidx_v, eidx_v, out_v,
              isem, isem2, rsem, osem):
    wid = lax.axis_index("s") * _NC + lax.axis_index("c")

    isems = [isem, isem2]

    def fire_ids(blk):
        pltpu.async_copy(stock_ids_hbm.at[pl.ds(blk * _BBLK, _BBLK)],
                         sidx_v.at[blk % 2], isems[blk % 2])
        pltpu.async_copy(sector_ids_hbm.at[pl.ds(blk * _BBLK, _BBLK)],
                         eidx_v.at[blk % 2], isems[blk % 2])

    def drain_ids(blk):
        pltpu.make_async_copy(stock_ids_hbm.at[pl.ds(0, _BBLK)],
                              sidx_v.at[blk % 2], isems[blk % 2]).wait()
        pltpu.make_async_copy(sector_ids_hbm.at[pl.ds(0, _BBLK)],
                              eidx_v.at[blk % 2], isems[blk % 2]).wait()

    def fire_row(d):
        return [pltpu.async_copy(stock_tabT_hbm.at[d], row_v, rsem),
                pltpu.async_copy(sector_tabT_hbm.at[d], secrow_v, rsem)]

    row_cps = fire_row(wid * _DPW)
    for t in range(_DPW):
        d = wid * _DPW + t
        fire_ids(0)
        ocps = []
        for blk in range(_NBLK):
            if blk + 1 < _NBLK:
                fire_ids(blk + 1)
            if blk == 0:
                for cp in row_cps:
                    cp.wait()
            drain_ids(blk)

            def gather32(i, carry):
                for u in range(2):
                    b0 = i * 32 + u * 16
                    ids16 = sidx_v[blk % 2, pl.ds(b0, 16)]
                    eids16 = eidx_v[blk % 2, pl.ds(b0, 16)]
                    svals = plsc.load_gather(row_v, [ids16])
                    evals = plsc.load_gather(secrow_v, [eids16])
                    out_v[blk % 2, pl.ds(b0, 16)] = svals + evals
                return carry

            lax.fori_loop(0, _BBLK // 32, gather32, 0)
            if blk >= 2:
                ocps[blk - 2].wait()
            ocps.append(pltpu.async_copy(
                out_v.at[blk % 2],
                outT_hbm.at[d, pl.ds(blk * _BBLK, _BBLK)], osem))
        if t + 1 < _DPW:
            row_cps = fire_row(d + 1)
        ocps[-2].wait()
        ocps[-1].wait()


def kernel(stock_ids, sector_ids, stock_table, sector_table):
    mesh = plsc.VectorSubcoreMesh(core_axis_name="c", subcore_axis_name="s")
    run = functools.partial(
        pl.kernel,
        mesh=mesh,
        out_type=jax.ShapeDtypeStruct((D_MODEL, BATCH), jnp.float32),
        scratch_types=[
            pltpu.VMEM((N_STOCKS,), jnp.float32),
            pltpu.VMEM((N_SECTORS,), jnp.float32),
            pltpu.VMEM((2, _BBLK), jnp.int32),
            pltpu.VMEM((2, _BBLK), jnp.int32),
            pltpu.VMEM((2, _BBLK), jnp.float32),
            pltpu.SemaphoreType.DMA,
            pltpu.SemaphoreType.DMA,
            pltpu.SemaphoreType.DMA,
            pltpu.SemaphoreType.DMA,
        ],
        compiler_params=pltpu.CompilerParams(needs_layout_passes=False),
    )(_emb_body)
    outT = run(stock_ids.astype(jnp.int32), sector_ids.astype(jnp.int32),
               stock_table.T, sector_table.T)
    return outT.T
